# Initial kernel scaffold; baseline (speedup 1.0000x reference)
import jax
import jax.numpy as jnp
from jax.experimental import pallas as pl


def kernel(points_coords, centers_coords, temb, points_features):
    raise NotImplementedError("write your pallas kernel here")


# speedup vs baseline: 10.2271x; 10.2271x over previous
import functools

import jax
import jax.numpy as jnp
import numpy as np
from jax.experimental import pallas as pl
from jax.experimental.pallas import tpu as pltpu
from jax.experimental.pallas import tpu_sc as plsc

_RADIUS = 0.1
_K = 32
_TM = 256
_TN = 2048


def _ballq_body(c_ref, p_ref, idx_ref, *, n_total, r2):
    b = pl.program_id(0)
    c = c_ref[0]
    cc = c[0] * c[0] + c[1] * c[1] + c[2] * c[2]
    tm = c.shape[1]
    n_chunks = n_total // _TN

    def chunk_body(i, carry):
        count, buf = carry
        p = p_ref[0, :, pl.ds(i * _TN, _TN)]
        pp = p[0] * p[0] + p[1] * p[1] + p[2] * p[2]
        cp = jax.lax.dot_general(
            c, p, (((0,), (0,)), ((), ())),
            preferred_element_type=jnp.float32)
        d2 = cc[:, None] + pp[None, :] - 2.0 * cp
        valid = d2 < r2
        vf = valid.astype(jnp.float32)
        cs = vf
        sh = 1
        while sh < _TN:
            z = jnp.zeros((tm, sh), jnp.float32)
            cs = cs + jnp.concatenate([z, cs[:, :-sh]], axis=1)
            sh *= 2
        rank = count[:, None] + (cs - vf).astype(jnp.int32)
        nglob = jax.lax.broadcasted_iota(jnp.int32, (tm, _TN), 1) + i * _TN
        kiota = jax.lax.broadcasted_iota(jnp.int32, (tm, _K), 1)
        newvals = jnp.full((tm, _K), -1, jnp.int32)
        for k in range(_K):
            hit = valid & (rank == k)
            v = jnp.max(jnp.where(hit, nglob, -1), axis=1)
            newvals = jnp.where(kiota == k, v[:, None], newvals)
        buf = jnp.where(newvals >= 0, newvals, buf)
        count = count + jnp.sum(valid.astype(jnp.int32), axis=1)
        return count, buf

    count0 = jnp.zeros((tm,), jnp.int32)
    buf0 = jnp.full((tm, _K), -1, jnp.int32)
    _, buf = jax.lax.fori_loop(0, n_chunks, chunk_body, (count0, buf0))
    first = buf[:, :1]
    first = jnp.where(first < 0, 0, first)
    idx = jnp.where(buf < 0, first, buf)
    idx_ref[0] = idx + b * n_total


def _ball_query(points_coords, centers_coords):
    B, _, N = points_coords.shape
    M = centers_coords.shape[2]
    r2 = np.float32(_RADIUS * _RADIUS)
    grid = (B, M // _TM)
    return pl.pallas_call(
        functools.partial(_ballq_body, n_total=N, r2=r2),
        grid=grid,
        in_specs=[
            pl.BlockSpec((1, 3, _TM), lambda b, m: (b, 0, m)),
            pl.BlockSpec((1, 3, N), lambda b, m: (b, 0, 0)),
        ],
        out_specs=pl.BlockSpec((1, _TM, _K), lambda b, m: (b, m, 0)),
        out_shape=jax.ShapeDtypeStruct((B, M, _K), jnp.int32),
        compiler_params=pltpu.CompilerParams(
            dimension_semantics=("parallel", "arbitrary")),
    )(centers_coords, points_coords)


def _sc_gather(data2d, idx_flat):
    num = idx_flat.shape[1]
    D = data2d.shape[1]
    W = 128
    mesh = plsc.VectorSubcoreMesh(core_axis_name="c", subcore_axis_name="s")

    @pl.kernel(
        out_type=jax.ShapeDtypeStruct((num, D), data2d.dtype),
        mesh=mesh,
    )
    def kern(x_hbm, i_hbm, o_hbm):
        def body(i_vmem, o_vmem):
            pltpu.sync_copy(x_hbm.at[i_vmem.at[0]], o_vmem)

        pltpu.emit_pipeline(
            body,
            grid=(num // W,),
            in_specs=[pl.BlockSpec((1, W), index_map=lambda i: (0, i))],
            out_specs=[pl.BlockSpec((W, D), index_map=lambda i: (i, 0))],
            core_axis_name=("c", "s"),
            dimension_semantics=(pltpu.PARALLEL,),
        )(i_hbm, o_hbm)

    return kern(data2d, idx_flat)


def kernel(points_coords, centers_coords, temb, points_features):
    B, _, N = points_coords.shape
    M = centers_coords.shape[2]
    C = points_features.shape[1]
    Ct = temb.shape[1]

    idx = _ball_query(points_coords, centers_coords)

    big = jnp.concatenate([points_coords, points_features, temb], axis=1)
    D = big.shape[1]
    Dp = 256
    big = jnp.pad(big, ((0, 0), (0, Dp - D), (0, 0)))
    data2d = big.transpose(0, 2, 1).reshape(B * N, Dp)

    idx_flat = idx.reshape(1, B * M * _K)
    gathered = _sc_gather(data2d, idx_flat)
    gathered = gathered.reshape(B, M, _K, Dp)

    coords = gathered[..., 0:3].transpose(0, 3, 1, 2)
    coords = coords - centers_coords[:, :, :, None]
    feats = gathered[..., 3:3 + C].transpose(0, 3, 1, 2)
    neighbor_features = jnp.concatenate([coords, feats], axis=1)
    grouped_temb = gathered[..., 3 + C:3 + C + Ct].transpose(0, 3, 1, 2)
    return (neighbor_features, grouped_temb)


# speedup vs baseline: 10.3192x; 1.0090x over previous
import functools

import jax
import jax.numpy as jnp
import numpy as np
from jax.experimental import pallas as pl
from jax.experimental.pallas import tpu as pltpu
from jax.experimental.pallas import tpu_sc as plsc

_RADIUS = 0.1
_K = 32
_TM = 256
_TN = 2048


def _ballq_body(c_ref, p_ref, idx_ref, *, n_total, r2):
    b = pl.program_id(0)
    c = c_ref[0]
    cc = c[0] * c[0] + c[1] * c[1] + c[2] * c[2]
    tm = c.shape[1]
    n_chunks = n_total // _TN

    def chunk_body(i, carry):
        count, buf = carry
        p = p_ref[0, :, pl.ds(i * _TN, _TN)]
        pp = p[0] * p[0] + p[1] * p[1] + p[2] * p[2]
        cp = jax.lax.dot_general(
            c, p, (((0,), (0,)), ((), ())),
            preferred_element_type=jnp.float32)
        d2 = cc[:, None] + pp[None, :] - 2.0 * cp
        valid = d2 < r2
        vf = valid.astype(jnp.float32)
        cs = vf
        sh = 1
        while sh < _TN:
            z = jnp.zeros((tm, sh), jnp.float32)
            cs = cs + jnp.concatenate([z, cs[:, :-sh]], axis=1)
            sh *= 2
        rank = count[:, None] + (cs - vf).astype(jnp.int32)
        nglob = jax.lax.broadcasted_iota(jnp.int32, (tm, _TN), 1) + i * _TN
        kiota = jax.lax.broadcasted_iota(jnp.int32, (tm, _K), 1)
        code = jnp.where(valid & (rank < _K), (rank << 14) | nglob, -1)
        cv = jnp.sum(valid.astype(jnp.int32), axis=1)
        count_new = count + cv
        active = (count < _K) & (cv > 0)
        k_lo = jnp.min(jnp.where(active, count, _K))
        k_hi = jnp.max(jnp.where(count < _K, jnp.minimum(count_new, _K), 0))

        def kbody(k, bufc):
            thresh = (k << 14) | 0x3FFF
            a = jnp.max(jnp.where(code <= thresh, code, -1), axis=1)
            upd = (kiota == k) & (a >= 0)[:, None]
            return jnp.where(upd, a[:, None], bufc)

        buf = jax.lax.fori_loop(k_lo, k_hi, kbody, buf)
        return count_new, buf

    count0 = jnp.zeros((tm,), jnp.int32)
    buf0 = jnp.full((tm, _K), -1, jnp.int32)
    count, buf = jax.lax.fori_loop(0, n_chunks, chunk_body, (count0, buf0))
    nbuf = buf & 0x3FFF
    first = jnp.where(count[:, None] > 0, nbuf[:, :1], 0)
    kiota2 = jax.lax.broadcasted_iota(jnp.int32, (tm, _K), 1)
    idx = jnp.where(kiota2 < count[:, None], nbuf, first)
    idx_ref[0] = idx + b * n_total


def _ball_query(points_coords, centers_coords):
    B, _, N = points_coords.shape
    M = centers_coords.shape[2]
    r2 = np.float32(_RADIUS * _RADIUS)
    grid = (B, M // _TM)
    return pl.pallas_call(
        functools.partial(_ballq_body, n_total=N, r2=r2),
        grid=grid,
        in_specs=[
            pl.BlockSpec((1, 3, _TM), lambda b, m: (b, 0, m)),
            pl.BlockSpec((1, 3, N), lambda b, m: (b, 0, 0)),
        ],
        out_specs=pl.BlockSpec((1, _TM, _K), lambda b, m: (b, m, 0)),
        out_shape=jax.ShapeDtypeStruct((B, M, _K), jnp.int32),
        compiler_params=pltpu.CompilerParams(
            dimension_semantics=("parallel", "arbitrary")),
    )(centers_coords, points_coords)


def _sc_gather(data2d, idx_flat):
    num = idx_flat.shape[1]
    D = data2d.shape[1]
    W = 128
    mesh = plsc.VectorSubcoreMesh(core_axis_name="c", subcore_axis_name="s")

    @pl.kernel(
        out_type=jax.ShapeDtypeStruct((num, D), data2d.dtype),
        mesh=mesh,
    )
    def kern(x_hbm, i_hbm, o_hbm):
        def body(i_vmem, o_vmem):
            pltpu.sync_copy(x_hbm.at[i_vmem.at[0]], o_vmem)

        pltpu.emit_pipeline(
            body,
            grid=(num // W,),
            in_specs=[pl.BlockSpec((1, W), index_map=lambda i: (0, i))],
            out_specs=[pl.BlockSpec((W, D), index_map=lambda i: (i, 0))],
            core_axis_name=("c", "s"),
            dimension_semantics=(pltpu.PARALLEL,),
        )(i_hbm, o_hbm)

    return kern(data2d, idx_flat)


def kernel(points_coords, centers_coords, temb, points_features):
    B, _, N = points_coords.shape
    M = centers_coords.shape[2]
    C = points_features.shape[1]
    Ct = temb.shape[1]

    idx = _ball_query(points_coords, centers_coords)

    big = jnp.concatenate([points_coords, points_features, temb], axis=1)
    D = big.shape[1]
    Dp = 256
    big = jnp.pad(big, ((0, 0), (0, Dp - D), (0, 0)))
    data2d = big.transpose(0, 2, 1).reshape(B * N, Dp)

    idx_flat = idx.reshape(1, B * M * _K)
    gathered = _sc_gather(data2d, idx_flat)
    gathered = gathered.reshape(B, M, _K, Dp)

    coords = gathered[..., 0:3].transpose(0, 3, 1, 2)
    coords = coords - centers_coords[:, :, :, None]
    feats = gathered[..., 3:3 + C].transpose(0, 3, 1, 2)
    neighbor_features = jnp.concatenate([coords, feats], axis=1)
    grouped_temb = gathered[..., 3 + C:3 + C + Ct].transpose(0, 3, 1, 2)
    return (neighbor_features, grouped_temb)
